# baseline (speedup 1.0000x reference)
import functools

import jax
import jax.numpy as jnp
from jax import lax
from jax.experimental import pallas as pl
from jax.experimental.pallas import tpu as pltpu
from jax.experimental.pallas import tpu_sc as plsc

NUM_PATTERNS = 1000000
PATTERN_DIM = 64
BATCH = 16384

NC = 2
NS = 16
NW = NC * NS
B_PER_W = BATCH // NW
CHUNK = 128
NCHUNK = B_PER_W // CHUNK


def _gather_body(tbl_hbm, idx_hbm, out_hbm, idx_v, rows_v, sem):
    wid = lax.axis_index("s") * NC + lax.axis_index("c")
    pltpu.sync_copy(idx_hbm.at[wid], idx_v)
    copies = [
        pltpu.async_copy(tbl_hbm.at[idx_v.at[j]], rows_v.at[j], sem)
        for j in range(NCHUNK)
    ]
    for cp in copies:
        cp.wait()
    pltpu.sync_copy(rows_v, out_hbm.at[wid])


@jax.jit
def _gather(patterns, idx):
    mesh = plsc.VectorSubcoreMesh(core_axis_name="c", subcore_axis_name="s")
    run = functools.partial(
        pl.kernel,
        mesh=mesh,
        out_type=jax.ShapeDtypeStruct((NW, NCHUNK, CHUNK, PATTERN_DIM),
                                      jnp.float32),
        scratch_types=[
            pltpu.VMEM((NCHUNK, CHUNK), jnp.int32),
            pltpu.VMEM((NCHUNK, CHUNK, PATTERN_DIM), jnp.float32),
            pltpu.SemaphoreType.DMA,
        ],
        compiler_params=pltpu.CompilerParams(use_tc_tiling_on_sc=False),
    )(_gather_body)
    out = run(patterns, idx.reshape(NW, NCHUNK, CHUNK))
    return out.reshape(BATCH, PATTERN_DIM)


def kernel(x, patterns):
    B = x.shape[0]
    idx = jax.random.randint(jax.random.key(42), (B,), 0,
                             patterns.shape[0], dtype=jnp.int32)
    return _gather(patterns, idx)


# speedup vs baseline: 2.1125x; 2.1125x over previous
import functools

import jax
import jax.numpy as jnp
from jax import lax
from jax.experimental import pallas as pl
from jax.experimental.pallas import tpu as pltpu
from jax.experimental.pallas import tpu_sc as plsc

NP = 1000000
D = 64
B = 16384

NC = 2
NS = 16
NW = NC * NS
CHUNK = 512
NFULL = NP // CHUNK
TAIL_LO = NFULL * CHUNK
TAIL_W = NP - TAIL_LO
CAP = 640
NVEC = B // 16
OUT1_W = NW * CAP


def _pass1_body(tblT, idx_hbm, out1, idxv, chunk, chunkT, colbuf, colj,
                bstart, sem_i, sem_c, sem_o):
    wid = lax.axis_index("s") * NC + lax.axis_index("c")
    is_w0 = wid == 0
    is_wlast = wid == NW - 1
    cstart = jnp.where(is_w0, 0, wid * 61 + 1)
    nchw = jnp.where(is_w0, 62, 61)
    lo_w = cstart * CHUNK
    hi_w = jnp.where(is_wlast, NP, (cstart + nchw) * CHUNK)

    pltpu.async_copy(idx_hbm, idxv, sem_i).wait()

    def scoped(cand_p, cand_j, bkt_p, bkt_j):
        iota = lax.iota(jnp.int32, 16)
        for k in range(CAP // 16):
            cand_p[pl.ds(16 * k, 16)] = jnp.full((16,), -1, jnp.int32)
            bkt_j[pl.ds(16 * k, 16)] = jnp.full((16,), B, jnp.int32)

        def scan(k, off):
            v = idxv[pl.ds(16 * k, 16)]
            m = (v >= lo_w) & (v < hi_w)
            plsc.store_compressed(cand_p.at[pl.ds(off, 16)], v, mask=m)
            plsc.store_compressed(cand_j.at[pl.ds(off, 16)], iota + 16 * k,
                                  mask=m)
            return off + plsc.all_reduce_population_count(m)[0]

        count = lax.fori_loop(0, NVEC, scan, 0)
        nvec_c = (count + 15) // 16

        nb = nchw + jnp.where(is_wlast, 1, 0)

        def bucket(i, boff):
            blo = (cstart + i) * CHUNK
            bhi = jnp.minimum(blo + CHUNK, NP)
            bstart[i] = boff

            def one(g, o):
                pv = cand_p[pl.ds(16 * g, 16)]
                m = (pv >= blo) & (pv < bhi)
                jv = cand_j[pl.ds(16 * g, 16)]
                plsc.store_compressed(bkt_p.at[pl.ds(o, 16)], pv, mask=m)
                plsc.store_compressed(bkt_j.at[pl.ds(o, 16)], jv, mask=m)
                return o + plsc.all_reduce_population_count(m)[0]

            return lax.fori_loop(0, nvec_c, one, boff)

        total = lax.fori_loop(0, nb, bucket, 0)
        bstart[nb] = total

        def gather_from(chunk_ref, lo_i, s0, s1):
            ngrp = (s1 - s0 + 15) // 16

            def grp(g, carry):
                base = s0 + 16 * g
                pv = bkt_p[pl.ds(base, 16)]
                m = iota < (s1 - base)
                lv = pv - lo_i
                slot = iota + base
                for d in range(D):
                    dfull = jnp.full((16,), d, jnp.int32)
                    vals = plsc.load_gather(chunk_ref, [dfull, lv], mask=m)
                    plsc.store_scatter(colbuf, [dfull, slot], vals, mask=m)
                return carry

            lax.fori_loop(0, ngrp, grp, 0)

        def run_chunk(i, carry):
            lo_i = pl.multiple_of((cstart + i) * CHUNK, 128)
            pltpu.async_copy(tblT.at[:, pl.ds(lo_i, CHUNK)], chunk,
                             sem_c).wait()
            gather_from(chunk, lo_i, bstart[i], bstart[i + 1])
            return carry

        lax.fori_loop(0, nchw, run_chunk, 0)

        @pl.when(is_wlast)
        def _tail():
            pltpu.async_copy(tblT.at[:, pl.ds(TAIL_LO, TAIL_W)], chunkT,
                             sem_c).wait()
            gather_from(chunkT, TAIL_LO, bstart[61], bstart[62])

        for k in range(CAP // 16):
            jv = bkt_j[pl.ds(16 * k, 16)]
            colj[0, pl.ds(16 * k, 16)] = plsc.bitcast(jv, jnp.float32)

    pl.run_scoped(scoped,
                  pltpu.VMEM((CAP,), jnp.int32),
                  pltpu.VMEM((CAP,), jnp.int32),
                  pltpu.VMEM((CAP,), jnp.int32),
                  pltpu.VMEM((CAP,), jnp.int32))

    wbase = pl.multiple_of(wid * CAP, 128)
    pltpu.async_copy(colbuf, out1.at[pl.ds(0, D), pl.ds(wbase, CAP)],
                     sem_o).wait()
    pltpu.async_copy(colj, out1.at[pl.ds(D, 8), pl.ds(wbase, CAP)],
                     sem_o).wait()


def _pass2_body(rows_hbm, slot_hbm, out_hbm, slot_v, rows_v, sem):
    wid = lax.axis_index("s") * NC + lax.axis_index("c")
    pltpu.sync_copy(slot_hbm.at[wid], slot_v)
    copies = [
        pltpu.async_copy(rows_hbm.at[slot_v.at[j]], rows_v.at[j], sem)
        for j in range(4)
    ]
    for cp in copies:
        cp.wait()
    pltpu.sync_copy(rows_v, out_hbm.at[wid])


@jax.jit
def _gather(patterns, idx):
    mesh = plsc.VectorSubcoreMesh(core_axis_name="c", subcore_axis_name="s")
    pass1 = functools.partial(
        pl.kernel,
        mesh=mesh,
        out_type=jax.ShapeDtypeStruct((D + 8, OUT1_W), jnp.float32),
        scratch_types=[
            pltpu.VMEM((B,), jnp.int32),
            pltpu.VMEM((D, CHUNK), jnp.float32),
            pltpu.VMEM((D, TAIL_W), jnp.float32),
            pltpu.VMEM((D, CAP), jnp.float32),
            pltpu.VMEM((8, CAP), jnp.float32),
            pltpu.SMEM((64,), jnp.int32),
            pltpu.SemaphoreType.DMA,
            pltpu.SemaphoreType.DMA,
            pltpu.SemaphoreType.DMA,
        ],
        compiler_params=pltpu.CompilerParams(use_tc_tiling_on_sc=True,
                                             needs_layout_passes=False),
    )(_pass1_body)
    out1 = pass1(patterns.T, idx)

    slot_map = lax.bitcast_convert_type(out1[D], jnp.int32)
    inv = jnp.zeros((B + 1,), jnp.int32).at[slot_map].set(
        jnp.arange(OUT1_W, dtype=jnp.int32))
    inv_slot = inv[:B]
    rows16k = out1[:D].T

    pass2 = functools.partial(
        pl.kernel,
        mesh=mesh,
        out_type=jax.ShapeDtypeStruct((NW, 4, 128, D), jnp.float32),
        scratch_types=[
            pltpu.VMEM((4, 128), jnp.int32),
            pltpu.VMEM((4, 128, D), jnp.float32),
            pltpu.SemaphoreType.DMA,
        ],
        compiler_params=pltpu.CompilerParams(use_tc_tiling_on_sc=False),
    )(_pass2_body)
    out = pass2(rows16k, inv_slot.reshape(NW, 4, 128))
    return out.reshape(B, D)


def kernel(x, patterns):
    idx = jax.random.randint(jax.random.key(42), (x.shape[0],), 0,
                             patterns.shape[0], dtype=jnp.int32)
    return _gather(patterns, idx)


# speedup vs baseline: 3.2078x; 1.5185x over previous
import functools

import jax
import jax.numpy as jnp
from jax import lax
from jax.experimental import pallas as pl
from jax.experimental.pallas import tpu as pltpu
from jax.experimental.pallas import tpu_sc as plsc

NP = 1000000
D = 64
B = 16384

NC = 2
NS = 16
NW = NC * NS
CHUNK = 512
NFULL = NP // CHUNK
TAIL_LO = NFULL * CHUNK
TAIL_W = NP - TAIL_LO
CAP = 640
NVEC = B // 16
OUT1_W = NW * CAP
JCH = CAP // 128


def _pass1_body(tblT, idx_hbm, out1, colbuf, colj, bstart,
                sem_i, sem_c0, sem_c1, sem_o):
    wid = lax.axis_index("s") * NC + lax.axis_index("c")
    is_w0 = wid == 0
    is_wlast = wid == NW - 1
    cstart = jnp.where(is_w0, 0, wid * 61 + 1)
    nchw = jnp.where(is_w0, 62, 61)
    lo_w = cstart * CHUNK
    hi_w = jnp.where(is_wlast, NP, (cstart + nchw) * CHUNK)
    iota = lax.iota(jnp.int32, 16)

    def scoped(cand_p, cand_j, bkt_p, bkt_j):
        for k in range(CAP // 16):
            cand_p[pl.ds(16 * k, 16)] = jnp.full((16,), -1, jnp.int32)
            bkt_j[pl.ds(16 * k, 16)] = B + iota + 16 * k

        def scope_idx(idxv):
            pltpu.async_copy(idx_hbm, idxv, sem_i).wait()

            def scan(k, off):
                v = idxv[pl.ds(16 * k, 16)]
                m = (v >= lo_w) & (v < hi_w)
                plsc.store_compressed(cand_p.at[pl.ds(off, 16)], v, mask=m)
                plsc.store_compressed(cand_j.at[pl.ds(off, 16)],
                                      iota + 16 * k, mask=m)
                return off + plsc.all_reduce_population_count(m)[0]

            bstart[63] = lax.fori_loop(0, NVEC, scan, 0)

        pl.run_scoped(scope_idx, pltpu.VMEM((B,), jnp.int32))
        count = bstart[63]
        nvec_c = (count + 15) // 16
        nb = nchw + jnp.where(is_wlast, 1, 0)

        def scope_stream(chunk, chunkT):
            def fire(i):
                lo_i = pl.multiple_of((cstart + i) * CHUNK, 128)
                src = tblT.at[:, pl.ds(lo_i, CHUNK)]

                @pl.when(i % 2 == 0)
                def _():
                    pltpu.make_async_copy(src, chunk.at[0], sem_c0).start()

                @pl.when(i % 2 == 1)
                def _():
                    pltpu.make_async_copy(src, chunk.at[1], sem_c1).start()

            fire(0)

            @pl.when(1 < nchw)
            def _():
                fire(1)

            def bucket(i, boff):
                blo = (cstart + i) * CHUNK
                bhi = jnp.minimum(blo + CHUNK, NP)
                bstart[i] = boff

                def one(g, o):
                    pv = cand_p[pl.ds(16 * g, 16)]
                    m = (pv >= blo) & (pv < bhi)
                    jv = cand_j[pl.ds(16 * g, 16)]
                    plsc.store_compressed(bkt_p.at[pl.ds(o, 16)], pv, mask=m)
                    plsc.store_compressed(bkt_j.at[pl.ds(o, 16)], jv, mask=m)
                    return o + plsc.all_reduce_population_count(m)[0]

                return lax.fori_loop(0, nvec_c, one, boff)

            total = lax.fori_loop(0, nb, bucket, 0)
            bstart[nb] = total

            def gather_from(chunk_ref, lo_i, s0, s1):
                ngrp = (s1 - s0 + 15) // 16

                def grp(g, carry):
                    base = s0 + 16 * g
                    pv = bkt_p[pl.ds(base, 16)]
                    m = iota < (s1 - base)
                    lv = pv - lo_i
                    slot = iota + base
                    for d in range(D):
                        dfull = jnp.full((16,), d, jnp.int32)
                        vals = plsc.load_gather(chunk_ref, [dfull, lv],
                                                mask=m)
                        plsc.store_scatter(colbuf, [dfull, slot], vals,
                                           mask=m)
                    return carry

                lax.fori_loop(0, ngrp, grp, 0)

            def run_chunk(i, carry):
                lo_i = pl.multiple_of((cstart + i) * CHUNK, 128)

                @pl.when(i % 2 == 0)
                def _():
                    pltpu.make_async_copy(tblT.at[:, pl.ds(0, CHUNK)],
                                          chunk.at[0], sem_c0).wait()
                    gather_from(chunk.at[0], lo_i, bstart[i], bstart[i + 1])

                @pl.when(i % 2 == 1)
                def _():
                    pltpu.make_async_copy(tblT.at[:, pl.ds(0, CHUNK)],
                                          chunk.at[1], sem_c1).wait()
                    gather_from(chunk.at[1], lo_i, bstart[i], bstart[i + 1])

                @pl.when(i + 2 < nchw)
                def _():
                    fire(i + 2)

                return carry

            lax.fori_loop(0, nchw, run_chunk, 0)

            @pl.when(is_wlast)
            def _tail():
                pltpu.async_copy(tblT.at[:, pl.ds(TAIL_LO, TAIL_W)], chunkT,
                                 sem_c0).wait()
                gather_from(chunkT, TAIL_LO, bstart[61], bstart[62])

        pl.run_scoped(scope_stream,
                      pltpu.VMEM((2, D, CHUNK), jnp.float32),
                      pltpu.VMEM((D, TAIL_W), jnp.float32))

        for k in range(CAP // 16):
            jv = bkt_j[pl.ds(16 * k, 16)]
            colj[0, pl.ds(16 * k, 16)] = plsc.bitcast(jv, jnp.float32)

    pl.run_scoped(scoped,
                  pltpu.VMEM((CAP,), jnp.int32),
                  pltpu.VMEM((CAP,), jnp.int32),
                  pltpu.VMEM((CAP,), jnp.int32),
                  pltpu.VMEM((CAP,), jnp.int32))

    wbase = pl.multiple_of(wid * CAP, 128)
    pltpu.async_copy(colbuf, out1.at[pl.ds(0, D), pl.ds(wbase, CAP)],
                     sem_o).wait()
    pltpu.async_copy(colj, out1.at[pl.ds(D, 8), pl.ds(wbase, CAP)],
                     sem_o).wait()


def _pass2_body(rows_hbm, jmap_hbm, out_hbm, jmap_v, rows_v, sem):
    wid = lax.axis_index("s") * NC + lax.axis_index("c")
    pltpu.sync_copy(jmap_hbm.at[wid], jmap_v)
    pltpu.sync_copy(rows_hbm.at[wid], rows_v)
    copies = [
        pltpu.async_copy(rows_v.at[k], out_hbm.at[jmap_v.at[k]], sem)
        for k in range(JCH)
    ]
    for cp in copies:
        cp.wait()


@jax.jit
def _gather(patterns, idx):
    mesh = plsc.VectorSubcoreMesh(core_axis_name="c", subcore_axis_name="s")
    pass1 = functools.partial(
        pl.kernel,
        mesh=mesh,
        out_type=jax.ShapeDtypeStruct((D + 8, OUT1_W), jnp.float32),
        scratch_types=[
            pltpu.VMEM((D, CAP), jnp.float32),
            pltpu.VMEM((8, CAP), jnp.float32),
            pltpu.SMEM((64,), jnp.int32),
            pltpu.SemaphoreType.DMA,
            pltpu.SemaphoreType.DMA,
            pltpu.SemaphoreType.DMA,
            pltpu.SemaphoreType.DMA,
        ],
        compiler_params=pltpu.CompilerParams(use_tc_tiling_on_sc=True,
                                             needs_layout_passes=False),
    )(_pass1_body)
    out1 = pass1(patterns.T, idx)

    slot_map = lax.bitcast_convert_type(out1[D], jnp.int32)
    rows16k = out1[:D].T

    pass2 = functools.partial(
        pl.kernel,
        mesh=mesh,
        out_type=jax.ShapeDtypeStruct((B + CAP, D), jnp.float32),
        scratch_types=[
            pltpu.VMEM((JCH, 128), jnp.int32),
            pltpu.VMEM((JCH, 128, D), jnp.float32),
            pltpu.SemaphoreType.DMA,
        ],
        compiler_params=pltpu.CompilerParams(use_tc_tiling_on_sc=False),
    )(_pass2_body)
    out = pass2(rows16k.reshape(NW, JCH, 128, D),
                slot_map.reshape(NW, JCH, 128))
    return out[:B]


def kernel(x, patterns):
    idx = jax.random.randint(jax.random.key(42), (x.shape[0],), 0,
                             patterns.shape[0], dtype=jnp.int32)
    return _gather(patterns, idx)
